# baseline (speedup 1.0000x reference)
import functools

import jax
import jax.numpy as jnp
from jax import lax
from jax.experimental import pallas as pl
from jax.experimental.pallas import tpu as pltpu

LANE = 128
_VMEM_LIMIT = 56 * 1024 * 1024
_TDN = (((1,), (1,)), ((), ()))
_NEG = float("-inf")


def _pad_lane(a, axis):
    pad = (-a.shape[axis]) % LANE
    if pad == 0:
        return a
    widths = [(0, 0)] * a.ndim
    widths[axis] = (0, pad)
    return jnp.pad(a, widths)


def _prep_sep(dw, pw, bn):
    gamma, beta, mean, var = bn
    scale = gamma / jnp.sqrt(var + 1e-5)
    shift = beta - mean * scale
    wdw = _pad_lane(jnp.transpose(dw[:, 0], (1, 2, 0)), 2)
    wpw = _pad_lane(_pad_lane(pw[:, :, 0, 0] * scale[:, None], 1), 0)
    wpw = wpw.astype(jnp.bfloat16)
    shift = _pad_lane(shift, 0)[None, :]
    return wdw, wpw, shift


def _dw3x3(read_kj, wdw, H, pre_relu):
    acc = None
    for kj in range(3):
        xj = read_kj(kj)
        if pre_relu:
            xj = jnp.maximum(xj, 0)
        for ki in range(3):
            term = xj[:, ki:ki + H] * wdw[ki, kj]
            acc = term if acc is None else acc + term
    return acc


def _exit_kernel(xh_ref,
                 wdw1_ref, wpw1_ref, sh1_ref,
                 wdw2_ref, wpw2_ref, sh2_ref,
                 wsc_ref, shs_ref,
                 wdw3_ref, wpw3_ref, sc3_ref, sh3_ref,
                 wdw4_ref, wpw4_ref, sc4_ref, sh4_ref,
                 o_ref, xpad_ref, apad_ref, mp_ref, pad3_ref, pad4_ref,
                 *, B, H, W):
    C0 = xh_ref.shape[-1]
    C2 = wpw2_ref.shape[0]
    C3 = wpw3_ref.shape[0]
    C4 = wpw4_ref.shape[0]
    Hp, Wp = H // 2, W // 2

    xpad_ref[...] = jnp.zeros(xpad_ref.shape, xpad_ref.dtype)
    xpad_ref[:, 1:H + 1, 1:W + 1, :] = xh_ref[...].astype(jnp.float32)

    wdw1 = wdw1_ref[...]
    acc = _dw3x3(lambda kj: xpad_ref[:, :, kj:kj + W, :],
                 wdw1, H, pre_relu=True)
    m1 = acc.reshape(B * H * W, C0).astype(jnp.bfloat16)
    h1 = lax.dot_general(m1, wpw1_ref[...], _TDN,
                         preferred_element_type=jnp.float32)
    h1 = jnp.maximum(h1 + sh1_ref[0], 0.0)

    apad_ref[...] = jnp.zeros(apad_ref.shape, apad_ref.dtype)
    apad_ref[:, 1:H + 1, 1:W + 1, :] = h1.reshape(B, H, W, C0)
    wdw2 = wdw2_ref[...]
    acc2 = _dw3x3(lambda kj: apad_ref[:, :, kj:kj + W, :],
                  wdw2, H, pre_relu=False)
    m2 = acc2.reshape(B * H * W, C0).astype(jnp.bfloat16)
    r = lax.dot_general(m2, wpw2_ref[...], _TDN,
                        preferred_element_type=jnp.float32)
    r = r + sh2_ref[0]

    xm = xh_ref[...].reshape(B, H, Wp, 2 * C0)[..., :C0]
    xs = xm.reshape(B, Hp, 2, Wp, C0)[:, :, 0]
    m_s = xs.reshape(B * Hp * Wp, C0)
    s = lax.dot_general(m_s, wsc_ref[...], _TDN,
                        preferred_element_type=jnp.float32)
    s = s + shs_ref[0]

    mp_ref[...] = jnp.full(mp_ref.shape, _NEG, mp_ref.dtype)
    mp_ref[:, 1:H + 1, 1:Wp + 1, :] = (
        r.reshape(B, H, Wp, 2 * C2).astype(jnp.bfloat16))
    mp = mp_ref[...]
    colmax = jnp.maximum(
        jnp.maximum(mp[:, :, 0:Wp, C2:], mp[:, :, 1:Wp + 1, :C2]),
        mp[:, :, 1:Wp + 1, C2:])
    a = colmax[:, 0:H].reshape(B, Hp, 2, Wp, C2)
    b = colmax[:, 2:H + 2].reshape(B, Hp, 2, Wp, C2)
    m = jnp.maximum(jnp.maximum(a[:, :, 0], a[:, :, 1]), b[:, :, 0])
    y = m.astype(jnp.float32) + s.reshape(B, Hp, Wp, C2)

    pad3_ref[...] = jnp.zeros(pad3_ref.shape, pad3_ref.dtype)
    pad3_ref[:, 1:Hp + 1, 1:Wp + 1, :] = y.astype(jnp.bfloat16)
    wdw3 = wdw3_ref[...]
    acc3 = _dw3x3(lambda kj: pad3_ref[:, :, kj:kj + Wp, :],
                  wdw3, Hp, pre_relu=False)
    m3 = acc3.reshape(B * Hp * Wp, C2).astype(jnp.bfloat16)
    h3 = lax.dot_general(m3, wpw3_ref[...].astype(jnp.bfloat16), _TDN,
                         preferred_element_type=jnp.float32)
    h3 = jnp.maximum(h3 * sc3_ref[0] + sh3_ref[0], 0.0)

    pad4_ref[...] = jnp.zeros(pad4_ref.shape, pad4_ref.dtype)
    pad4_ref[:, 1:Hp + 1, 1:Wp + 1, :] = h3.reshape(B, Hp, Wp, C3).astype(jnp.bfloat16)
    wdw4 = wdw4_ref[...]
    acc4 = _dw3x3(lambda kj: pad4_ref[:, :, kj:kj + Wp, :],
                  wdw4, Hp, pre_relu=False)
    m4 = acc4.reshape(B * Hp * Wp, C3).astype(jnp.bfloat16)
    h4 = lax.dot_general(m4, wpw4_ref[...].astype(jnp.bfloat16), _TDN,
                         preferred_element_type=jnp.float32)
    h4 = jnp.maximum(h4 * sc4_ref[0] + sh4_ref[0], 0.0)
    o_ref[:, 0, :] = jnp.mean(h4.reshape(B, Hp * Wp, C4), axis=1)


def kernel(x, dw1, pw1, bn1_g, bn1_b, bn1_m, bn1_v,
           dw2, pw2, bn2_g, bn2_b, bn2_m, bn2_v,
           w_sc, bnsc_g, bnsc_b, bnsc_m, bnsc_v,
           dw3, pw3, bn3_g, bn3_b, bn3_m, bn3_v,
           dw4, pw4, bn4_g, bn4_b, bn4_m, bn4_v):
    N, _, H, W = x.shape
    dt = x.dtype
    c_out = pw4.shape[0]

    wdw1, wpw1, sh1 = _prep_sep(dw1, pw1, (bn1_g, bn1_b, bn1_m, bn1_v))
    wdw2, wpw2, sh2 = _prep_sep(dw2, pw2, (bn2_g, bn2_b, bn2_m, bn2_v))
    wdw3 = jnp.transpose(dw3[:, 0], (1, 2, 0))
    wdw4 = jnp.transpose(dw4[:, 0], (1, 2, 0))
    wpw3 = pw3[:, :, 0, 0]
    wpw4 = pw4[:, :, 0, 0]
    sc3 = (bn3_g / jnp.sqrt(bn3_v + 1e-5))[None, :]
    sh3 = (bn3_b - bn3_m * sc3[0])[None, :]
    sc4 = (bn4_g / jnp.sqrt(bn4_v + 1e-5))[None, :]
    sh4 = (bn4_b - bn4_m * sc4[0])[None, :]
    scs = bnsc_g / jnp.sqrt(bnsc_v + 1e-5)
    shs = _pad_lane(bnsc_b - bnsc_m * scs, 0)[None, :]
    wsc = _pad_lane(_pad_lane(w_sc[:, :, 0, 0] * scs[:, None], 1), 0)
    wsc = wsc.astype(jnp.bfloat16)

    xh = _pad_lane(jnp.transpose(x, (0, 2, 3, 1)), 3).astype(jnp.bfloat16)
    C0 = xh.shape[-1]
    C2 = wpw2.shape[0]
    C3 = wpw3.shape[0]
    C4 = wpw4.shape[0]
    Hp, Wp = H // 2, W // 2

    BA = 4 if N % 4 == 0 else (2 if N % 2 == 0 else 1)
    kfn = functools.partial(_exit_kernel, B=BA, H=H, W=W)
    cost = pl.CostEstimate(
        flops=2 * N * H * W * C0 * (2 * C0 + C2 + 18)
              + 2 * N * Hp * Wp * (C0 * C2 + C2 * C3 + C3 * C4
                                   + 9 * (C2 + C3)),
        transcendentals=0,
        bytes_accessed=int(xh.size * 2 + N * C4 * 4
                           + 2 * (wpw1.size + wpw2.size + wsc.size
                                  + wpw3.size + wpw4.size)))
    o = pl.pallas_call(
        kfn,
        out_shape=jax.ShapeDtypeStruct((N, 1, C4), jnp.float32),
        grid=(N // BA,),
        in_specs=[
            pl.BlockSpec((BA, H, W, C0), lambda i: (i, 0, 0, 0)),
            pl.BlockSpec((3, 3, C0), lambda i: (0, 0, 0)),
            pl.BlockSpec((C0, C0), lambda i: (0, 0)),
            pl.BlockSpec((1, C0), lambda i: (0, 0)),
            pl.BlockSpec((3, 3, C0), lambda i: (0, 0, 0)),
            pl.BlockSpec((C2, C0), lambda i: (0, 0)),
            pl.BlockSpec((1, C2), lambda i: (0, 0)),
            pl.BlockSpec((C2, C0), lambda i: (0, 0)),
            pl.BlockSpec((1, C2), lambda i: (0, 0)),
            pl.BlockSpec((3, 3, C2), lambda i: (0, 0, 0)),
            pl.BlockSpec((C3, C2), lambda i: (0, 0)),
            pl.BlockSpec((1, C3), lambda i: (0, 0)),
            pl.BlockSpec((1, C3), lambda i: (0, 0)),
            pl.BlockSpec((3, 3, C3), lambda i: (0, 0, 0)),
            pl.BlockSpec((C4, C3), lambda i: (0, 0)),
            pl.BlockSpec((1, C4), lambda i: (0, 0)),
            pl.BlockSpec((1, C4), lambda i: (0, 0)),
        ],
        out_specs=pl.BlockSpec((BA, 1, C4), lambda i: (i, 0, 0)),
        scratch_shapes=[
            pltpu.VMEM((BA, H + 2, W + 2, C0), jnp.float32),
            pltpu.VMEM((BA, H + 2, W + 2, C0), jnp.float32),
            pltpu.VMEM((BA, H + 2, Wp + 2, 2 * C2), jnp.bfloat16),
            pltpu.VMEM((BA, Hp + 2, Wp + 2, C2), jnp.bfloat16),
            pltpu.VMEM((BA, Hp + 2, Wp + 2, C3), jnp.bfloat16),
        ],
        compiler_params=pltpu.CompilerParams(
            dimension_semantics=("parallel",),
            vmem_limit_bytes=_VMEM_LIMIT),
        cost_estimate=cost,
    )(xh, wdw1, wpw1, sh1, wdw2, wpw2, sh2, wsc, shs,
      wdw3, wpw3, sc3, sh3, wdw4, wpw4, sc4, sh4)

    return o.reshape(N, C4)[:, :c_out].astype(dt)[:, :, None, None]


# speedup vs baseline: 1.1360x; 1.1360x over previous
import functools

import jax
import jax.numpy as jnp
from jax import lax
from jax.experimental import pallas as pl
from jax.experimental.pallas import tpu as pltpu

LANE = 128
_VMEM_LIMIT = 56 * 1024 * 1024
_TDN = (((1,), (1,)), ((), ()))
_NEG = float("-inf")


def _pad_lane(a, axis):
    pad = (-a.shape[axis]) % LANE
    if pad == 0:
        return a
    widths = [(0, 0)] * a.ndim
    widths[axis] = (0, pad)
    return jnp.pad(a, widths)


def _prep_sep(dw, pw, bn):
    gamma, beta, mean, var = bn
    scale = gamma / jnp.sqrt(var + 1e-5)
    shift = beta - mean * scale
    wdw = _pad_lane(jnp.transpose(dw[:, 0], (1, 2, 0)), 2)
    wpw = _pad_lane(_pad_lane(pw[:, :, 0, 0] * scale[:, None], 1), 0)
    wpw = wpw.astype(jnp.bfloat16)
    shift = _pad_lane(shift, 0)[None, :]
    return wdw, wpw, shift


def _dw3x3(read_kj, wdw, H, pre_relu):
    acc = None
    for kj in range(3):
        xj = read_kj(kj)
        if pre_relu:
            xj = jnp.maximum(xj, 0)
        for ki in range(3):
            term = xj[:, ki:ki + H] * wdw[ki, kj]
            acc = term if acc is None else acc + term
    return acc


def _exit_kernel(xh_ref,
                 wdw1_ref, wpw1_ref, sh1_ref,
                 wdw2_ref, wpw2_ref, sh2_ref,
                 wsc_ref, shs_ref,
                 wdw3_ref, wpw3_ref, sh3_ref,
                 wdw4_ref, wpw4_ref, sh4_ref,
                 o_ref, xpad_ref, apad_ref, mp_ref, pad3_ref, pad4_ref,
                 *, B, H, W):
    C0 = xh_ref.shape[-1]
    C2 = wpw2_ref.shape[0]
    C3 = wpw3_ref.shape[0]
    C4 = wpw4_ref.shape[0]
    Hp, Wp = H // 2, W // 2

    xpad_ref[...] = jnp.zeros(xpad_ref.shape, xpad_ref.dtype)
    xpad_ref[:, 1:H + 1, 1:W + 1, :] = xh_ref[...].astype(jnp.float32)

    wdw1 = wdw1_ref[...]
    acc = _dw3x3(lambda kj: xpad_ref[:, :, kj:kj + W, :],
                 wdw1, H, pre_relu=True)
    m1 = acc.reshape(B * H * W, C0).astype(jnp.bfloat16)
    h1 = lax.dot_general(m1, wpw1_ref[...], _TDN,
                         preferred_element_type=jnp.float32)
    h1 = jnp.maximum(h1 + sh1_ref[0], 0.0)

    apad_ref[...] = jnp.zeros(apad_ref.shape, apad_ref.dtype)
    apad_ref[:, 1:H + 1, 1:W + 1, :] = h1.reshape(B, H, W, C0)
    wdw2 = wdw2_ref[...]
    acc2 = _dw3x3(lambda kj: apad_ref[:, :, kj:kj + W, :],
                  wdw2, H, pre_relu=False)
    m2 = acc2.reshape(B * H * W, C0).astype(jnp.bfloat16)
    r = lax.dot_general(m2, wpw2_ref[...], _TDN,
                        preferred_element_type=jnp.float32)
    r = r + sh2_ref[0]

    xm = xh_ref[...].reshape(B, H, Wp, 2 * C0)[..., :C0]
    xs = xm.reshape(B, Hp, 2, Wp, C0)[:, :, 0]
    m_s = xs.reshape(B * Hp * Wp, C0)
    s = lax.dot_general(m_s, wsc_ref[...], _TDN,
                        preferred_element_type=jnp.float32)
    s = s + shs_ref[0]

    mp_ref[...] = jnp.full(mp_ref.shape, _NEG, mp_ref.dtype)
    mp_ref[:, 1:H + 1, 1:Wp + 1, :] = (
        r.reshape(B, H, Wp, 2 * C2).astype(jnp.bfloat16))
    mp = mp_ref[...]
    colmax = jnp.maximum(
        jnp.maximum(mp[:, :, 0:Wp, C2:], mp[:, :, 1:Wp + 1, :C2]),
        mp[:, :, 1:Wp + 1, C2:])
    a = colmax[:, 0:H].reshape(B, Hp, 2, Wp, C2)
    b = colmax[:, 2:H + 2].reshape(B, Hp, 2, Wp, C2)
    m = jnp.maximum(jnp.maximum(a[:, :, 0], a[:, :, 1]), b[:, :, 0])
    y = m.astype(jnp.float32) + s.reshape(B, Hp, Wp, C2)

    pad3_ref[...] = jnp.zeros(pad3_ref.shape, pad3_ref.dtype)
    pad3_ref[:, 1:Hp + 1, 1:Wp + 1, :] = y.astype(jnp.bfloat16)
    wdw3 = wdw3_ref[...]
    acc3 = _dw3x3(lambda kj: pad3_ref[:, :, kj:kj + Wp, :],
                  wdw3, Hp, pre_relu=False)
    m3 = acc3.reshape(B * Hp * Wp, C2).astype(jnp.bfloat16)
    h3 = lax.dot_general(m3, wpw3_ref[...], _TDN,
                         preferred_element_type=jnp.float32)
    h3 = jnp.maximum(h3 + sh3_ref[0], 0.0)

    pad4_ref[...] = jnp.zeros(pad4_ref.shape, pad4_ref.dtype)
    pad4_ref[:, 1:Hp + 1, 1:Wp + 1, :] = h3.reshape(B, Hp, Wp, C3).astype(jnp.bfloat16)
    wdw4 = wdw4_ref[...]
    acc4 = _dw3x3(lambda kj: pad4_ref[:, :, kj:kj + Wp, :],
                  wdw4, Hp, pre_relu=False)
    m4 = acc4.reshape(B * Hp * Wp, C3).astype(jnp.bfloat16)
    h4 = lax.dot_general(m4, wpw4_ref[...], _TDN,
                         preferred_element_type=jnp.float32)
    h4 = jnp.maximum(h4 + sh4_ref[0], 0.0)
    o_ref[:, 0, :] = jnp.mean(h4.reshape(B, Hp * Wp, C4), axis=1)


def kernel(x, dw1, pw1, bn1_g, bn1_b, bn1_m, bn1_v,
           dw2, pw2, bn2_g, bn2_b, bn2_m, bn2_v,
           w_sc, bnsc_g, bnsc_b, bnsc_m, bnsc_v,
           dw3, pw3, bn3_g, bn3_b, bn3_m, bn3_v,
           dw4, pw4, bn4_g, bn4_b, bn4_m, bn4_v):
    N, _, H, W = x.shape
    dt = x.dtype
    c_out = pw4.shape[0]

    wdw1, wpw1, sh1 = _prep_sep(dw1, pw1, (bn1_g, bn1_b, bn1_m, bn1_v))
    wdw2, wpw2, sh2 = _prep_sep(dw2, pw2, (bn2_g, bn2_b, bn2_m, bn2_v))
    wdw3, wpw3, sh3 = _prep_sep(dw3, pw3, (bn3_g, bn3_b, bn3_m, bn3_v))
    wdw4, wpw4, sh4 = _prep_sep(dw4, pw4, (bn4_g, bn4_b, bn4_m, bn4_v))
    scs = bnsc_g / jnp.sqrt(bnsc_v + 1e-5)
    shs = _pad_lane(bnsc_b - bnsc_m * scs, 0)[None, :]
    wsc = _pad_lane(_pad_lane(w_sc[:, :, 0, 0] * scs[:, None], 1), 0)
    wsc = wsc.astype(jnp.bfloat16)

    xh = _pad_lane(jnp.transpose(x, (0, 2, 3, 1)), 3).astype(jnp.bfloat16)
    C0 = xh.shape[-1]
    C2 = wpw2.shape[0]
    C3 = wpw3.shape[0]
    C4 = wpw4.shape[0]
    Hp, Wp = H // 2, W // 2

    BA = 4 if N % 4 == 0 else (2 if N % 2 == 0 else 1)
    kfn = functools.partial(_exit_kernel, B=BA, H=H, W=W)
    cost = pl.CostEstimate(
        flops=2 * N * H * W * C0 * (2 * C0 + C2 + 18)
              + 2 * N * Hp * Wp * (C0 * C2 + C2 * C3 + C3 * C4
                                   + 9 * (C2 + C3)),
        transcendentals=0,
        bytes_accessed=int(xh.size * 2 + N * C4 * 4
                           + 2 * (wpw1.size + wpw2.size + wsc.size
                                  + wpw3.size + wpw4.size)))
    o = pl.pallas_call(
        kfn,
        out_shape=jax.ShapeDtypeStruct((N, 1, C4), jnp.float32),
        grid=(N // BA,),
        in_specs=[
            pl.BlockSpec((BA, H, W, C0), lambda i: (i, 0, 0, 0)),
            pl.BlockSpec((3, 3, C0), lambda i: (0, 0, 0)),
            pl.BlockSpec((C0, C0), lambda i: (0, 0)),
            pl.BlockSpec((1, C0), lambda i: (0, 0)),
            pl.BlockSpec((3, 3, C0), lambda i: (0, 0, 0)),
            pl.BlockSpec((C2, C0), lambda i: (0, 0)),
            pl.BlockSpec((1, C2), lambda i: (0, 0)),
            pl.BlockSpec((C2, C0), lambda i: (0, 0)),
            pl.BlockSpec((1, C2), lambda i: (0, 0)),
            pl.BlockSpec((3, 3, C2), lambda i: (0, 0, 0)),
            pl.BlockSpec((C3, C2), lambda i: (0, 0)),
            pl.BlockSpec((1, C3), lambda i: (0, 0)),
            pl.BlockSpec((3, 3, C3), lambda i: (0, 0, 0)),
            pl.BlockSpec((C4, C3), lambda i: (0, 0)),
            pl.BlockSpec((1, C4), lambda i: (0, 0)),
        ],
        out_specs=pl.BlockSpec((BA, 1, C4), lambda i: (i, 0, 0)),
        scratch_shapes=[
            pltpu.VMEM((BA, H + 2, W + 2, C0), jnp.float32),
            pltpu.VMEM((BA, H + 2, W + 2, C0), jnp.float32),
            pltpu.VMEM((BA, H + 2, Wp + 2, 2 * C2), jnp.bfloat16),
            pltpu.VMEM((BA, Hp + 2, Wp + 2, C2), jnp.bfloat16),
            pltpu.VMEM((BA, Hp + 2, Wp + 2, C3), jnp.bfloat16),
        ],
        compiler_params=pltpu.CompilerParams(
            dimension_semantics=("parallel",),
            vmem_limit_bytes=_VMEM_LIMIT),
        cost_estimate=cost,
    )(xh, wdw1, wpw1, sh1, wdw2, wpw2, sh2, wsc, shs,
      wdw3, wpw3, sh3, wdw4, wpw4, sh4)

    return o.reshape(N, C4)[:, :c_out].astype(dt)[:, :, None, None]
